# baseline (speedup 1.0000x reference)
import functools

import jax
import jax.numpy as jnp
from jax import lax
from jax.experimental import pallas as pl
from jax.experimental.pallas import tpu as pltpu
from jax.experimental.pallas import tpu_sc as plsc

VOCAB = 1000000
D = 64
BT = 4096
S = 200
NC = 2
NS = 16
NW = NC * NS
BPW = BT // NW
SP = 208
NBUF = 4
SB = 13

_mesh = plsc.VectorSubcoreMesh(core_axis_name="c", subcore_axis_name="s")


@functools.partial(
    pl.kernel,
    out_type=jax.ShapeDtypeStruct((BT, S, D), jnp.float32),
    mesh=_mesh,
    scratch_types=[
        pltpu.VMEM((S // 8, 8 * BPW), jnp.int32),
        pltpu.VMEM((BPW, SP), jnp.int32),
        pltpu.VMEM((NBUF, S, D), jnp.float32),
        [pltpu.SemaphoreType.DMA] * NBUF,
        [pltpu.SemaphoreType.DMA] * NBUF,
        [pltpu.SemaphoreType.DMA] * NBUF,
    ],
    compiler_params=pltpu.CompilerParams(use_tc_tiling_on_sc=False,
                                         needs_layout_passes=False),
)
def _emb(idx_hbm, table_hbm, out_hbm, idx_raw, idx_t, bufs, gsa, gsb, oss):
    wid = lax.axis_index("s") * NC + lax.axis_index("c")

    pltpu.sync_copy(idx_hbm.at[:, wid], idx_raw)

    lanes = jnp.arange(16, dtype=jnp.int32)

    def transpose_col(c, _=None):
        cv = jnp.zeros((16,), jnp.int32) + c
        for k in range(SB):
            s = jnp.minimum(16 * k + lanes, S - 1)
            vals = plsc.load_gather(idx_raw, [s >> 3, ((s & 7) << 7) + cv])
            idx_t[c, pl.ds(16 * k, 16)] = vals

    pl.loop(0, BPW)(transpose_col)

    def start_gather(c, b):
        pltpu.async_copy(table_hbm.at[idx_t.at[c, pl.ds(0, 128)]],
                         bufs.at[b, pl.ds(0, 128)], gsa[b])
        pltpu.async_copy(table_hbm.at[idx_t.at[c, pl.ds(128, S - 128)]],
                         bufs.at[b, pl.ds(128, S - 128)], gsb[b])

    def wait_gather(c, b):
        pltpu.make_async_copy(table_hbm.at[idx_t.at[c, pl.ds(0, 128)]],
                              bufs.at[b, pl.ds(0, 128)], gsa[b]).wait()
        pltpu.make_async_copy(table_hbm.at[idx_t.at[c, pl.ds(128, S - 128)]],
                              bufs.at[b, pl.ds(128, S - 128)], gsb[b]).wait()

    def start_out(c, b):
        pltpu.async_copy(bufs.at[b], out_hbm.at[wid * BPW + c], oss[b])

    def wait_out(c, b):
        pltpu.make_async_copy(bufs.at[b], out_hbm.at[wid * BPW + c],
                              oss[b]).wait()

    for b in range(NBUF):
        start_gather(b, b)

    def ring(j, _=None):
        for b in range(NBUF):
            c = NBUF * j + b
            wait_gather(c, b)
            start_out(c, b)
            wait_out(c, b)
            start_gather(c + NBUF, b)

    pl.loop(0, (BPW - NBUF) // NBUF)(ring)

    for b in range(NBUF):
        c = BPW - NBUF + b
        wait_gather(c, b)
        start_out(c, b)
        wait_out(c, b)


def kernel(indices, table):
    x = indices.astype(jnp.int32).reshape(NW, BPW, S // 8, 8)
    x = x.transpose(2, 0, 3, 1).reshape(S // 8, NW, 8 * BPW)
    return _emb(x, table)
